# baseline (speedup 1.0000x reference)
import jax
import jax.numpy as jnp
from jax import lax
from jax.experimental import pallas as pl
from jax.experimental.pallas import tpu as pltpu
from jax.experimental.pallas import tpu_sc as plsc

_NC, _NS = 2, 16
_NW = _NC * _NS
_N = 100000
_G = 512
_D = 128
_ROWS_PER_W = _N // _NW
_CH = 125
_NCH = _ROWS_PER_W // _CH


def _sc_segsum_body(nodes, idx3, zeros, partial, shared, nbuf, idxv, sems):
    c = lax.axis_index("c")
    s = lax.axis_index("s")
    wid = c * _NS + s
    base = wid * _ROWS_PER_W
    gpt = _G // _NS

    pltpu.sync_copy(zeros.at[pl.ds(s * gpt, gpt)], shared.at[pl.ds(s * gpt, gpt)])
    pltpu.sync_copy(idx3.at[wid], idxv)
    plsc.subcore_barrier()

    def stage(j, p):
        pltpu.async_copy(
            nodes.at[pl.ds(base + j * _CH, _CH), :], nbuf.at[p], sems.at[p]
        )

    stage(0, 0)

    @pl.loop(0, _NCH)
    def _(j):
        p = lax.rem(j, 2)

        @pl.when(j + 1 < _NCH)
        def _():
            stage(j + 1, lax.rem(j + 1, 2))

        pltpu.make_async_copy(
            nodes.at[pl.ds(base, _CH), :], nbuf.at[p], sems.at[p]
        ).wait()
        pltpu.sync_copy(nbuf.at[p], shared.at[idxv.at[j]], add=True)

    plsc.subcore_barrier()
    pltpu.sync_copy(
        shared.at[pl.ds(s * gpt, gpt)], partial.at[c, pl.ds(s * gpt, gpt)]
    )


def _combine_body(partial_ref, glob_ref, w_ref, b_ref, out_ref):
    segsum = jnp.sum(partial_ref[...], axis=0)
    d_feat = segsum.shape[1]
    w_top = w_ref[:d_feat, :]
    w_bot = w_ref[d_feat:, :]
    out_ref[...] = (
        jnp.dot(segsum, w_top, preferred_element_type=jnp.float32)
        + jnp.dot(glob_ref[...], w_bot, preferred_element_type=jnp.float32)
        + b_ref[...][None, :]
    )


@jax.jit
def kernel(nodes, edges, receivers, senders, global_latent, node_graph_idx,
           edge_graph_idx, W, b):
    n_graphs, d_global = global_latent.shape
    d_out = W.shape[1]
    idx3 = node_graph_idx.reshape(_NW, _NCH, _CH)
    zeros = jnp.zeros((_G, _D), jnp.float32)

    mesh = plsc.VectorSubcoreMesh(core_axis_name="c", subcore_axis_name="s")
    sc_segsum = pl.kernel(
        _sc_segsum_body,
        out_type=jax.ShapeDtypeStruct((_NC, _G, _D), jnp.float32),
        mesh=mesh,
        scratch_types=[
            pltpu.VMEM_SHARED((_G, _D), jnp.float32),
            pltpu.VMEM((2, _CH, _D), jnp.float32),
            pltpu.VMEM((_NCH, _CH), jnp.int32),
            pltpu.SemaphoreType.DMA((2,)),
        ],
        compiler_params=pltpu.CompilerParams(use_tc_tiling_on_sc=False),
    )
    partial = sc_segsum(nodes, idx3, zeros)

    return pl.pallas_call(
        _combine_body,
        out_shape=jax.ShapeDtypeStruct((n_graphs, d_out), jnp.float32),
    )(partial, global_latent, W, b)


# speedup vs baseline: 1.0125x; 1.0125x over previous
import jax
import jax.numpy as jnp
from jax import lax
from jax.experimental import pallas as pl
from jax.experimental.pallas import tpu as pltpu
from jax.experimental.pallas import tpu_sc as plsc

_NC, _NS = 2, 16
_NW = _NC * _NS
_N = 100000
_G = 512
_D = 128
_ROWS_PER_W = _N // _NW
_CH = 125
_NCH = _ROWS_PER_W // _CH


_D_RING = 5


def _sc_segsum_body(nodes, idx3, partial, shared, nbuf, idxv, zbuf, sem_s, sem_c):
    c = lax.axis_index("c")
    s = lax.axis_index("s")
    wid = c * _NS + s
    base = wid * _ROWS_PER_W
    gpt = _G // _NS

    @pl.loop(0, gpt)
    def _(i):
        @pl.loop(0, _D // 16)
        def _(k):
            zbuf[i, pl.ds(k * 16, 16)] = jnp.zeros((16,), jnp.float32)

    pltpu.sync_copy(zbuf, shared.at[pl.ds(s * gpt, gpt)])
    pltpu.sync_copy(idx3.at[wid], idxv)

    def stage(j, b):
        pltpu.async_copy(
            nodes.at[pl.ds(base + j * _CH, _CH), :], nbuf.at[b], sem_s.at[b]
        )

    for b in range(_D_RING):
        stage(b, b)

    plsc.subcore_barrier()

    @pl.loop(0, _NCH // _D_RING)
    def _(g):
        j0 = g * _D_RING
        for b in range(_D_RING):
            pltpu.make_async_copy(
                nodes.at[pl.ds(base, _CH), :], nbuf.at[b], sem_s.at[b]
            ).wait()
            pltpu.async_copy(
                nbuf.at[b], shared.at[idxv.at[j0 + b]], sem_c.at[b], add=True
            )
        for b in range(_D_RING):
            pltpu.make_async_copy(
                nbuf.at[b], shared.at[idxv.at[0]], sem_c.at[b]
            ).wait()

            @pl.when(j0 + _D_RING + b < _NCH)
            def _():
                stage(j0 + _D_RING + b, b)

    plsc.subcore_barrier()
    pltpu.sync_copy(
        shared.at[pl.ds(s * gpt, gpt)], partial.at[c, pl.ds(s * gpt, gpt)]
    )


def _combine_body(partial_ref, glob_ref, w_ref, b_ref, out_ref):
    segsum = jnp.sum(partial_ref[...], axis=0)
    d_feat = segsum.shape[1]
    w_top = w_ref[:d_feat, :]
    w_bot = w_ref[d_feat:, :]
    out_ref[...] = (
        jnp.dot(segsum, w_top, preferred_element_type=jnp.float32)
        + jnp.dot(glob_ref[...], w_bot, preferred_element_type=jnp.float32)
        + b_ref[...][None, :]
    )


@jax.jit
def kernel(nodes, edges, receivers, senders, global_latent, node_graph_idx,
           edge_graph_idx, W, b):
    n_graphs, d_global = global_latent.shape
    d_out = W.shape[1]
    idx3 = node_graph_idx.reshape(_NW, _NCH, _CH)

    mesh = plsc.VectorSubcoreMesh(core_axis_name="c", subcore_axis_name="s")
    sc_segsum = pl.kernel(
        _sc_segsum_body,
        out_type=jax.ShapeDtypeStruct((_NC, _G, _D), jnp.float32),
        mesh=mesh,
        scratch_types=[
            pltpu.VMEM_SHARED((_G, _D), jnp.float32),
            pltpu.VMEM((_D_RING, _CH, _D), jnp.float32),
            pltpu.VMEM((_NCH, _CH), jnp.int32),
            pltpu.VMEM((_G // _NS, _D), jnp.float32),
            pltpu.SemaphoreType.DMA((_D_RING,)),
            pltpu.SemaphoreType.DMA((_D_RING,)),
        ],
        compiler_params=pltpu.CompilerParams(use_tc_tiling_on_sc=False),
    )
    partial = sc_segsum(nodes, idx3)

    return pl.pallas_call(
        _combine_body,
        out_shape=jax.ShapeDtypeStruct((n_graphs, d_out), jnp.float32),
    )(partial, global_latent, W, b)
